# baseline (speedup 1.0000x reference)
import functools

import jax
import jax.numpy as jnp
from jax import lax
from jax.experimental import pallas as pl
from jax.experimental.pallas import tpu as pltpu
from jax.experimental.pallas import tpu_sc as plsc

BATCH = 16384
EMBED_DIM = 768
NUM_CLASSES = 12
NUM_CORES = 2
NUM_SUBCORES = 16
NUM_WORKERS = NUM_CORES * NUM_SUBCORES
ROWS_PER_WORKER = BATCH // NUM_WORKERS
CHUNK = 64
NUM_CHUNKS = ROWS_PER_WORKER // CHUNK
LANES = 16
GROUPS = EMBED_DIM // LANES
J_UNROLL = 2


@functools.partial(
    pl.kernel,
    mesh=plsc.VectorSubcoreMesh(core_axis_name="c", subcore_axis_name="s"),
    out_type=jax.ShapeDtypeStruct((BATCH, EMBED_DIM), jnp.float32),
    scratch_types=[
        pltpu.VMEM((NUM_CLASSES, EMBED_DIM), jnp.float32),
        pltpu.VMEM((ROWS_PER_WORKER,), jnp.int32),
        pltpu.VMEM((CHUNK, EMBED_DIM), jnp.float32),
        pltpu.VMEM((CHUNK, EMBED_DIM), jnp.float32),
        pltpu.SemaphoreType.DMA,
        pltpu.SemaphoreType.DMA,
    ],
    compiler_params=pltpu.CompilerParams(needs_layout_passes=False),
)
def _lookup(labels_hbm, table_hbm, out_hbm, table_v, idx_v,
            buf0, buf1, wsem0, wsem1):
  wid = lax.axis_index("s") * NUM_CORES + lax.axis_index("c")
  base = wid * ROWS_PER_WORKER
  bufs = (buf0, buf1)
  wsems = (wsem0, wsem1)

  pltpu.sync_copy(table_hbm, table_v)
  pltpu.sync_copy(labels_hbm.at[pl.ds(base, ROWS_PER_WORKER)], idx_v)

  writes = [None, None]
  for k in range(NUM_CHUNKS):
    b = k % 2
    if k >= 2:
      writes[b].wait()
    buf = bufs[b]

    @plsc.parallel_loop(0, CHUNK // LANES, 1, unroll=2)
    def grp_body(g, k=k, buf=buf):
      labv = idx_v[pl.ds(k * CHUNK + g * LANES, LANES)]
      iota = lax.iota(jnp.int32, LANES)
      labs = [labv[l] for l in range(LANES)]
      rowvs = [iota * 0 + (g * LANES + l) for l in range(LANES)]

      @plsc.parallel_loop(0, GROUPS, 1, unroll=J_UNROLL)
      def j_body(j, labs=labs, rowvs=rowvs, iota=iota, buf=buf):
        colv = iota + j * LANES
        for l in range(LANES):
          val = table_v[labs[l], pl.ds(j * LANES, LANES)]
          plsc.store_scatter(buf, [rowvs[l], colv], val)
    writes[b] = pltpu.async_copy(
        buf, out_hbm.at[pl.ds(base + k * CHUNK, CHUNK)], wsems[b])
  writes[0].wait()
  writes[1].wait()


def kernel(labels, table):
  return _lookup(labels.astype(jnp.int32), table)


# speedup vs baseline: 1.0887x; 1.0887x over previous
import functools

import jax
import jax.numpy as jnp
from jax import lax
from jax.experimental import pallas as pl
from jax.experimental.pallas import tpu as pltpu
from jax.experimental.pallas import tpu_sc as plsc

BATCH = 16384
EMBED_DIM = 768
NUM_CLASSES = 12
NUM_CORES = 2
NUM_SUBCORES = 16
NUM_WORKERS = NUM_CORES * NUM_SUBCORES
ROWS_PER_WORKER = BATCH // NUM_WORKERS
CHUNK = 64
NUM_CHUNKS = ROWS_PER_WORKER // CHUNK
LANES = 16
GROUPS = EMBED_DIM // LANES
J_UNROLL = 2


@functools.partial(
    pl.kernel,
    mesh=plsc.VectorSubcoreMesh(core_axis_name="c", subcore_axis_name="s"),
    out_type=jax.ShapeDtypeStruct((BATCH, EMBED_DIM), jnp.float32),
    scratch_types=[
        pltpu.VMEM((NUM_CLASSES, EMBED_DIM), jnp.float32),
        pltpu.VMEM((ROWS_PER_WORKER,), jnp.int32),
        pltpu.VMEM((CHUNK, EMBED_DIM), jnp.float32),
        pltpu.VMEM((CHUNK, EMBED_DIM), jnp.float32),
        pltpu.SemaphoreType.DMA,
        pltpu.SemaphoreType.DMA,
        pltpu.SemaphoreType.DMA,
        pltpu.SemaphoreType.DMA,
    ],
    compiler_params=pltpu.CompilerParams(needs_layout_passes=False),
)
def _lookup(labels_hbm, table_hbm, out_hbm, table_v, idx_v,
            buf0, buf1, wsem0, wsem1, ssem0, ssem1):
  wid = lax.axis_index("s") * NUM_CORES + lax.axis_index("c")
  base = wid * ROWS_PER_WORKER
  bufs = (buf0, buf1)
  wsems = (wsem0, wsem1)

  st = pltpu.async_copy(table_hbm, table_v, ssem0)
  sl = pltpu.async_copy(labels_hbm.at[pl.ds(base, ROWS_PER_WORKER)], idx_v,
                        ssem1)
  st.wait()
  sl.wait()

  writes = [None, None]
  for k in range(NUM_CHUNKS):
    b = k % 2
    if k >= 2:
      writes[b].wait()
    buf = bufs[b]

    @plsc.parallel_loop(0, CHUNK // LANES, 1)
    def grp_body(g, k=k, buf=buf):
      labv = idx_v[pl.ds(k * CHUNK + g * LANES, LANES)]
      iota = lax.iota(jnp.int32, LANES)
      labs = [labv[l] for l in range(LANES)]
      rowvs = [iota * 0 + (g * LANES + l) for l in range(LANES)]

      @plsc.parallel_loop(0, GROUPS, 1, unroll=J_UNROLL)
      def j_body(j, labs=labs, rowvs=rowvs, iota=iota, buf=buf):
        colv = iota + j * LANES
        for l in range(LANES):
          val = table_v[labs[l], pl.ds(j * LANES, LANES)]
          plsc.store_scatter(buf, [rowvs[l], colv], val)
    writes[b] = pltpu.async_copy(
        buf, out_hbm.at[pl.ds(base + k * CHUNK, CHUNK)], wsems[b])
  writes[0].wait()
  writes[1].wait()


def kernel(labels, table):
  return _lookup(labels.astype(jnp.int32), table)
